# Initial kernel scaffold; baseline (speedup 1.0000x reference)
import jax
import jax.numpy as jnp
from jax.experimental import pallas as pl


def kernel(x_nchw, w1, w2):
    raise NotImplementedError("write your pallas kernel here")


# speedup vs baseline: 1.6117x; 1.6117x over previous
import functools

import jax
import jax.numpy as jnp
from jax.experimental import pallas as pl
from jax.experimental.pallas import tpu as pltpu


def _se_folded_kernel(x_ref, w1_ref, w2_ref, o_ref, *, hw, k, inv_hw):
    x = x_ref[...]
    xf = x.astype(jnp.float32)
    r = x.shape[1]
    segs = [jnp.sum(xf[:, :, j * hw:(j + 1) * hw], axis=2) for j in range(k)]
    pooled = jnp.concatenate(segs, axis=1) * inv_hw
    y1 = jnp.dot(pooled, w1_ref[...], preferred_element_type=jnp.float32)
    y1 = jnp.maximum(y1, 0.0)
    g = jax.nn.sigmoid(
        jnp.dot(y1, w2_ref[...], preferred_element_type=jnp.float32))
    lane = jax.lax.broadcasted_iota(jnp.int32, x.shape, dimension=2)
    scale = g[:, 0:r][:, :, None]
    for j in range(1, k):
        scale = jnp.where(lane >= j * hw, g[:, j * r:(j + 1) * r][:, :, None],
                          scale)
    o_ref[...] = (xf * scale).astype(o_ref.dtype)


def _se_general_kernel(x_ref, w1_ref, w2_ref, o_ref, *, inv_hw):
    x = x_ref[...].astype(jnp.float32)
    pooled = jnp.sum(x, axis=2) * inv_hw
    y1 = jnp.dot(pooled, w1_ref[...], preferred_element_type=jnp.float32)
    y1 = jnp.maximum(y1, 0.0)
    g = jax.nn.sigmoid(
        jnp.dot(y1, w2_ref[...], preferred_element_type=jnp.float32))
    o_ref[...] = (x * g[:, :, None]).astype(o_ref.dtype)


def _pick_bt(b):
    for cand in (16, 8, 32, 64, 4, 2, 1):
        if b % cand == 0 and b // cand >= 2:
            return cand
    return b


@functools.partial(jax.jit, static_argnames=())
def _se_block(x_nchw, w1, w2):
    B, C, H, W = x_nchw.shape
    HW = H * W
    Cr = w1.shape[0]

    w1t = jnp.transpose(w1).astype(jnp.float32)
    w2t = jnp.transpose(w2).astype(jnp.float32)

    fold = (HW < 128) and (128 % HW == 0) and ((C * HW) % 128 == 0)
    bt = _pick_bt(B)
    nb = B // bt

    cost = pl.CostEstimate(
        flops=int(3 * B * C * HW + 4 * B * C * Cr),
        transcendentals=int(B * C),
        bytes_accessed=int(2 * B * C * HW * x_nchw.dtype.itemsize
                           + 2 * C * Cr * 4),
    )

    if fold:
        k = 128 // HW
        R = (C * HW) // 128
        x_f = x_nchw.reshape(B, R, 128)
        perm = jnp.arange(C).reshape(R, k).T.reshape(-1)
        w1p = w1t[perm, :]
        w2p = w2t[:, perm]
        out = pl.pallas_call(
            functools.partial(_se_folded_kernel, hw=HW, k=k, inv_hw=1.0 / HW),
            out_shape=jax.ShapeDtypeStruct((B, R, 128), x_nchw.dtype),
            grid=(nb,),
            in_specs=[
                pl.BlockSpec((bt, R, 128), lambda b: (b, 0, 0)),
                pl.BlockSpec((C, Cr), lambda b: (0, 0)),
                pl.BlockSpec((Cr, C), lambda b: (0, 0)),
            ],
            out_specs=pl.BlockSpec((bt, R, 128), lambda b: (b, 0, 0)),
            compiler_params=pltpu.CompilerParams(
                dimension_semantics=("parallel",)),
            cost_estimate=cost,
        )(x_f, w1p, w2p)
    else:
        x3 = x_nchw.reshape(B, C, HW)
        out = pl.pallas_call(
            functools.partial(_se_general_kernel, inv_hw=1.0 / HW),
            out_shape=jax.ShapeDtypeStruct((B, C, HW), x_nchw.dtype),
            grid=(nb,),
            in_specs=[
                pl.BlockSpec((bt, C, HW), lambda b: (b, 0, 0)),
                pl.BlockSpec((C, Cr), lambda b: (0, 0)),
                pl.BlockSpec((Cr, C), lambda b: (0, 0)),
            ],
            out_specs=pl.BlockSpec((bt, C, HW), lambda b: (b, 0, 0)),
            compiler_params=pltpu.CompilerParams(
                dimension_semantics=("parallel",)),
            cost_estimate=cost,
        )(x3, w1t, w2t)

    return out.reshape(B, C, H, W)


def kernel(x_nchw, w1, w2):
    return _se_block(x_nchw, w1, w2)


# speedup vs baseline: 10.1584x; 6.3031x over previous
import functools

import jax
import jax.numpy as jnp
from jax.experimental import pallas as pl
from jax.experimental.pallas import tpu as pltpu


def _se_kernel(x_ref, w1_ref, w2_ref, o_ref, *, inv_hw):
    x = x_ref[...].astype(jnp.float32)
    pooled = jnp.sum(x, axis=1) * inv_hw
    y1 = jnp.dot(pooled, w1_ref[...], preferred_element_type=jnp.float32)
    y1 = jnp.maximum(y1, 0.0)
    g = jax.nn.sigmoid(
        jnp.dot(y1, w2_ref[...], preferred_element_type=jnp.float32))
    o_ref[...] = (x * g[:, None, :]).astype(o_ref.dtype)


def _pick_bt(b):
    for cand in (16, 8, 32, 64, 4, 2, 1):
        if b % cand == 0 and b // cand >= 2:
            return cand
    return b


@jax.jit
def _se_block(x_nchw, w1, w2):
    B, C, H, W = x_nchw.shape
    HW = H * W
    Cr = w1.shape[0]

    w1t = jnp.transpose(w1).astype(jnp.float32)
    w2t = jnp.transpose(w2).astype(jnp.float32)

    x_v = jnp.transpose(x_nchw, (0, 2, 3, 1)).reshape(B, HW, C)

    bt = _pick_bt(B)
    nb = B // bt

    cost = pl.CostEstimate(
        flops=int(3 * B * C * HW + 4 * B * C * Cr),
        transcendentals=int(B * C),
        bytes_accessed=int(2 * B * C * HW * x_nchw.dtype.itemsize
                           + 2 * C * Cr * 4),
    )

    out = pl.pallas_call(
        functools.partial(_se_kernel, inv_hw=1.0 / HW),
        out_shape=jax.ShapeDtypeStruct((B, HW, C), x_nchw.dtype),
        grid=(nb,),
        in_specs=[
            pl.BlockSpec((bt, HW, C), lambda b: (b, 0, 0)),
            pl.BlockSpec((C, Cr), lambda b: (0, 0)),
            pl.BlockSpec((Cr, C), lambda b: (0, 0)),
        ],
        out_specs=pl.BlockSpec((bt, HW, C), lambda b: (b, 0, 0)),
        compiler_params=pltpu.CompilerParams(
            dimension_semantics=("parallel",)),
        cost_estimate=cost,
    )(x_v, w1t, w2t)

    return jnp.transpose(out.reshape(B, H, W, C), (0, 3, 1, 2))


def kernel(x_nchw, w1, w2):
    return _se_block(x_nchw, w1, w2)
